# baseline (speedup 1.0000x reference)
import jax
import jax.numpy as jnp
from jax import lax
from jax.experimental import pallas as pl
from jax.experimental.pallas import tpu as pltpu
from jax.experimental.pallas import tpu_sc as plsc

N = 10000
E = 320000
D = 128
H = 128
NL = 4
DE = 2

NSL = 2
ES = E // NSL

NC = 2
NS = 16
NW = NC * NS
CH = 80
NCHUNK = ES // CH
MAXC = 64
NCPAD = MAXC * NW

ZR = 80
NZC = N // ZR
ZPW = -(-NZC // NS)

BN = 2000
BE = 2000

_f32 = jnp.float32
_i32 = jnp.int32
_mesh = plsc.VectorSubcoreMesh(core_axis_name="c", subcore_axis_name="s")


def _dot(a, b):
    return jnp.dot(a, b, preferred_element_type=_f32)


def _silu(x):
    return x * (0.5 + 0.5 * jnp.tanh(0.5 * x))


def _embed_body(h_ref, w_ref, b_ref, ws_ref, wd_ref, o_ref, os_ref, od_ref):
    x = _dot(h_ref[...], w_ref[...]) + b_ref[...]
    o_ref[...] = x
    os_ref[...] = _dot(x, ws_ref[...])
    od_ref[...] = _dot(x, wd_ref[...])


def _edge_body(g1_ref, g2_ref, ea_ref, wea_ref, b1_ref, w2_ref, b2_ref, o_ref):
    ea = ea_ref[...]
    pre = (g1_ref[...] + g2_ref[...] + b1_ref[...]
           + ea[:, 0:1] * wea_ref[0:1, :] + ea[:, 1:2] * wea_ref[1:2, :])
    m = _silu(pre)
    o_ref[...] = _silu(_dot(m, w2_ref[...]) + b2_ref[...])


def _node_mid_body(x_ref, pa_ref, pb_ref, wx_ref, wa_ref, b1_ref, w2_ref,
                   b2_ref, ws_ref, wd_ref, o_ref, os_ref, od_ref):
    x = x_ref[...]
    agg = pa_ref[0] + pa_ref[1] + pb_ref[0] + pb_ref[1]
    t = _silu(_dot(x, wx_ref[...]) + _dot(agg, wa_ref[...]) + b1_ref[...])
    xn = x + _dot(t, w2_ref[...]) + b2_ref[...]
    o_ref[...] = xn
    os_ref[...] = _dot(xn, ws_ref[...])
    od_ref[...] = _dot(xn, wd_ref[...])


def _node_last_body(x_ref, pa_ref, pb_ref, wx_ref, wa_ref, b1_ref, w2_ref,
                    b2_ref, d1_ref, db1_ref, d2_ref, db2_ref, o_ref):
    x = x_ref[...]
    agg = pa_ref[0] + pa_ref[1] + pb_ref[0] + pb_ref[1]
    t = _silu(_dot(x, wx_ref[...]) + _dot(agg, wa_ref[...]) + b1_ref[...])
    xn = x + _dot(t, w2_ref[...]) + b2_ref[...]
    u = _silu(_dot(xn, d1_ref[...]) + db1_ref[...])
    o_ref[...] = _dot(u, d2_ref[...]) + db2_ref[...]


def _row_block(bn):
    return pl.BlockSpec((bn, H), lambda i: (i, 0))


def _full(shape):
    return pl.BlockSpec(shape, lambda i: tuple(0 for _ in shape))


def _embed(h, w, b, ws, wd):
    return pl.pallas_call(
        _embed_body,
        grid=(N // BN,),
        in_specs=[pl.BlockSpec((BN, D), lambda i: (i, 0)),
                  _full((D, H)), _full((1, H)), _full((H, H)), _full((H, H))],
        out_specs=[_row_block(BN), _row_block(BN), _row_block(BN)],
        out_shape=[jax.ShapeDtypeStruct((N, H), _f32)] * 3,
    )(h, w, b.reshape(1, H), ws, wd)


def _edge_mlp(g, ea, wea, b1, w2, b2):
    return pl.pallas_call(
        _edge_body,
        grid=(ES // BE,),
        in_specs=[_row_block(BE), _row_block(BE),
                  pl.BlockSpec((BE, DE), lambda i: (i, 0)),
                  _full((DE, H)), _full((1, H)), _full((H, H)), _full((1, H))],
        out_specs=_row_block(BE),
        out_shape=jax.ShapeDtypeStruct((ES, H), _f32),
    )(g[0], g[1], ea, wea, b1.reshape(1, H), w2, b2.reshape(1, H))


def _parts_specs():
    return [pl.BlockSpec((NC, BN, H), lambda i: (0, i, 0)),
            pl.BlockSpec((NC, BN, H), lambda i: (0, i, 0))]


def _node_mid(x, pa, pb, wx, wa, b1, w2, b2, ws, wd):
    return pl.pallas_call(
        _node_mid_body,
        grid=(N // BN,),
        in_specs=[_row_block(BN)] + _parts_specs() +
                 [_full((H, H)), _full((H, H)), _full((1, H)),
                  _full((H, H)), _full((1, H)), _full((H, H)), _full((H, H))],
        out_specs=[_row_block(BN), _row_block(BN), _row_block(BN)],
        out_shape=[jax.ShapeDtypeStruct((N, H), _f32)] * 3,
    )(x, pa, pb, wx, wa, b1.reshape(1, H), w2, b2.reshape(1, H), ws, wd)


def _node_last(x, pa, pb, wx, wa, b1, w2, b2, d1W, d1b, d2W, d2b):
    d2p = jnp.zeros((H, 128), _f32).at[:, :3].set(d2W)
    db2p = jnp.zeros((1, 128), _f32).at[0, :3].set(d2b)
    y = pl.pallas_call(
        _node_last_body,
        grid=(N // BN,),
        in_specs=[_row_block(BN)] + _parts_specs() +
                 [_full((H, H)), _full((H, H)), _full((1, H)),
                  _full((H, H)), _full((1, H)), _full((H, H)), _full((1, H)),
                  _full((H, 128)), _full((1, 128))],
        out_specs=pl.BlockSpec((BN, 128), lambda i: (i, 0)),
        out_shape=jax.ShapeDtypeStruct((N, 128), _f32),
    )(x, pa, pb, wx, wa, b1.reshape(1, H), w2, b2.reshape(1, H),
      d1W, d1b.reshape(1, H), d2p, db2p)
    return y[:, :3]


def _worker_range(wid):
    c0 = wid * MAXC
    cnt = jnp.minimum(MAXC, NCHUNK - c0)
    return c0, cnt


def _sc_gather_body(xs_hbm, xd_hbm, row_hbm, col_hbm, g1_hbm, g2_hbm,
                    ridx, cidx, b1a, b1b, b2a, b2b,
                    s1a, s1b, s2a, s2b, w1a, w1b, w2a, w2b):
    cid = lax.axis_index("c")
    sid = lax.axis_index("s")
    wid = sid * NC + cid
    c0, cnt = _worker_range(wid)
    pltpu.sync_copy(row_hbm.at[pl.ds(c0, MAXC)], ridx)
    pltpu.sync_copy(col_hbm.at[pl.ds(c0, MAXC)], cidx)

    def wbs(k, buf1, buf2, sw1, sw2):
        base = (c0 + k) * CH
        return (pltpu.make_async_copy(buf1, g1_hbm.at[pl.ds(base, CH)], sw1),
                pltpu.make_async_copy(buf2, g2_hbm.at[pl.ds(base, CH)], sw2))

    def fire(k, buf1, buf2, sem1, sem2, sw1, sw2):
        @pl.when(k < cnt)
        def _():
            @pl.when(k >= 2)
            def _():
                cp1, cp2 = wbs(k - 2, buf1, buf2, sw1, sw2)
                cp1.wait()
                cp2.wait()

            pltpu.make_async_copy(xs_hbm.at[ridx.at[k]], buf1, sem1).start()
            pltpu.make_async_copy(xd_hbm.at[cidx.at[k]], buf2, sem2).start()

    def drain(k, buf1, buf2, sem1, sem2, sw1, sw2):
        @pl.when(k < cnt)
        def _():
            pltpu.make_async_copy(xs_hbm.at[ridx.at[k]], buf1, sem1).wait()
            pltpu.make_async_copy(xd_hbm.at[cidx.at[k]], buf2, sem2).wait()
            cp1, cp2 = wbs(k, buf1, buf2, sw1, sw2)
            cp1.start()
            cp2.start()

    fire(0, b1a, b2a, s1a, s2a, w1a, w2a)

    @pl.loop(0, (MAXC + 1) // 2)
    def _(j):
        k = 2 * j
        fire(k + 1, b1b, b2b, s1b, s2b, w1b, w2b)
        drain(k, b1a, b2a, s1a, s2a, w1a, w2a)
        fire(k + 2, b1a, b2a, s1a, s2a, w1a, w2a)
        drain(k + 1, b1b, b2b, s1b, s2b, w1b, w2b)

    @pl.when(cnt >= 2)
    def _():
        cp1, cp2 = wbs(cnt - 2, b1a, b2a, w1a, w2a)
        cp1.wait()
        cp2.wait()

    @pl.when(cnt >= 1)
    def _():
        cp1, cp2 = wbs(cnt - 1, b1b, b2b, w1b, w2b)
        cp1.wait()
        cp2.wait()


def _sc_gather(xs, xd, row2d, col2d):
    k = pl.kernel(
        _sc_gather_body,
        out_type=[jax.ShapeDtypeStruct((ES, H), _f32),
                  jax.ShapeDtypeStruct((ES, H), _f32)],
        mesh=_mesh,
        scratch_types=[pltpu.VMEM((MAXC, CH), _i32),
                       pltpu.VMEM((MAXC, CH), _i32),
                       pltpu.VMEM((CH, H), _f32),
                       pltpu.VMEM((CH, H), _f32),
                       pltpu.VMEM((CH, H), _f32),
                       pltpu.VMEM((CH, H), _f32),
                       pltpu.SemaphoreType.DMA,
                       pltpu.SemaphoreType.DMA,
                       pltpu.SemaphoreType.DMA,
                       pltpu.SemaphoreType.DMA,
                       pltpu.SemaphoreType.DMA,
                       pltpu.SemaphoreType.DMA,
                       pltpu.SemaphoreType.DMA,
                       pltpu.SemaphoreType.DMA],
    )
    return k(xs, xd, row2d, col2d)


def _sc_scatter_body(m2_hbm, row_hbm, out_hbm, ridx, bufa, bufb, zbuf, acc,
                     sema, semb, semz):
    cid = lax.axis_index("c")
    sid = lax.axis_index("s")
    wid = sid * NC + cid
    c0, cnt = _worker_range(wid)

    pltpu.sync_copy(row_hbm.at[pl.ds(c0, MAXC)], ridx)

    @pl.loop(0, ZR)
    def _(r):
        @pl.loop(0, H // 16)
        def _(j):
            zbuf[r, pl.ds(j * 16, 16)] = jnp.zeros((16,), _f32)

    def zcopy(j):
        z = sid + j * NS
        return z < NZC, pltpu.make_async_copy(zbuf, acc.at[pl.ds(z * ZR, ZR)],
                                              semz)

    @pl.loop(0, ZPW)
    def _(j):
        ok, cp = zcopy(j)

        @pl.when(ok)
        def _():
            cp.start()

    @pl.loop(0, ZPW)
    def _(j):
        ok, cp = zcopy(j)

        @pl.when(ok)
        def _():
            cp.wait()

    plsc.subcore_barrier()

    def fire(k, buf, sem):
        @pl.when(k < cnt)
        def _():
            base = (c0 + k) * CH
            pltpu.make_async_copy(m2_hbm.at[pl.ds(base, CH)], buf, sem).start()

    def drain(k, buf, sem):
        @pl.when(k < cnt)
        def _():
            base = (c0 + k) * CH
            pltpu.make_async_copy(m2_hbm.at[pl.ds(base, CH)], buf, sem).wait()
            pltpu.sync_copy(buf, acc.at[ridx.at[k]], add=True)

    fire(0, bufa, sema)

    @pl.loop(0, (MAXC + 1) // 2)
    def _(j):
        k = 2 * j
        fire(k + 1, bufb, semb)
        drain(k, bufa, sema)
        fire(k + 2, bufa, sema)
        drain(k + 1, bufb, semb)

    plsc.subcore_barrier()

    def ocopy(j):
        z = sid + j * NS
        r0 = z * ZR
        return z < NZC, pltpu.make_async_copy(
            acc.at[pl.ds(r0, ZR)], out_hbm.at[cid, pl.ds(r0, ZR)], semz)

    @pl.loop(0, ZPW)
    def _(j):
        ok, cp = ocopy(j)

        @pl.when(ok)
        def _():
            cp.start()

    @pl.loop(0, ZPW)
    def _(j):
        ok, cp = ocopy(j)

        @pl.when(ok)
        def _():
            cp.wait()


def _sc_scatter(m2, row2d):
    k = pl.kernel(
        _sc_scatter_body,
        out_type=jax.ShapeDtypeStruct((NC, N, H), _f32),
        mesh=_mesh,
        scratch_types=[pltpu.VMEM((MAXC, CH), _i32),
                       pltpu.VMEM((CH, H), _f32),
                       pltpu.VMEM((CH, H), _f32),
                       pltpu.VMEM((ZR, H), _f32),
                       pltpu.VMEM_SHARED((N, H), _f32),
                       pltpu.SemaphoreType.DMA,
                       pltpu.SemaphoreType.DMA,
                       pltpu.SemaphoreType.DMA],
    )
    return k(m2, row2d)


def _slice_chunks(idx1d):
    c = idx1d.reshape(NSL, NCHUNK, CH)
    pad = ((0, NCPAD - NCHUNK), (0, 0))
    return [jnp.pad(c[s], pad) for s in range(NSL)]


def kernel(h, edges, edge_attr, emb_W, emb_b, e1W, e1b, e2W, e2b,
           n1W, n1b, n2W, n2b, d1W, d1b, d2W, d2b):
    rows = _slice_chunks(edges[0])
    cols = _slice_chunks(edges[1])
    eas = [edge_attr[s * ES:(s + 1) * ES] for s in range(NSL)]

    x, xs, xd = _embed(h, emb_W, emb_b, e1W[0, :H], e1W[0, H:2 * H])
    for i in range(NL):
        wea = e1W[i, 2 * H:]
        gs = [_sc_gather(xs, xd, rows[s], cols[s]) for s in range(NSL)]
        m2s = [_edge_mlp(gs[s], eas[s], wea, e1b[i], e2W[i], e2b[i])
               for s in range(NSL)]
        parts = [_sc_scatter(m2s[s], rows[s]) for s in range(NSL)]
        if i < NL - 1:
            x, xs, xd = _node_mid(x, parts[0], parts[1], n1W[i, :H],
                                  n1W[i, H:], n1b[i], n2W[i], n2b[i],
                                  e1W[i + 1, :H], e1W[i + 1, H:2 * H])
        else:
            return _node_last(x, parts[0], parts[1], n1W[i, :H], n1W[i, H:],
                              n1b[i], n2W[i], n2b[i], d1W, d1b, d2W, d2b)


# speedup vs baseline: 1.1868x; 1.1868x over previous
import jax
import jax.numpy as jnp
from jax import lax
from jax.experimental import pallas as pl
from jax.experimental.pallas import tpu as pltpu
from jax.experimental.pallas import tpu_sc as plsc

N = 10000
E = 320000
D = 128
H = 128
NL = 4
DE = 2

NSL = 2
ES = E // NSL

NC = 2
NS = 16
NW = NC * NS
CH = 80
NCHUNK = ES // CH
MAXC = 64
NCPAD = MAXC * NW

CHS = 128
NCHUNKS = ES // CHS
MAXCS = 40
NCPADS = MAXCS * NW

ZR = 80
NZC = N // ZR
ZPW = -(-NZC // NS)

BN = 2000
BE = 2000

_f32 = jnp.float32
_i32 = jnp.int32
_mesh = plsc.VectorSubcoreMesh(core_axis_name="c", subcore_axis_name="s")


def _dot(a, b):
    return jnp.dot(a, b, preferred_element_type=_f32)


def _silu(x):
    return x * (0.5 + 0.5 * jnp.tanh(0.5 * x))


def _embed_body(h_ref, w_ref, b_ref, ws_ref, wd_ref, o_ref, os_ref, od_ref):
    x = _dot(h_ref[...], w_ref[...]) + b_ref[...]
    o_ref[...] = x
    os_ref[...] = _dot(x, ws_ref[...])
    od_ref[...] = _dot(x, wd_ref[...])


def _edge_body(g_ref, ea_ref, wea_ref, b1_ref, w2_ref, b2_ref, o_ref):
    ea = ea_ref[...]
    pre = (g_ref[...] + b1_ref[...]
           + ea[:, 0:1] * wea_ref[0:1, :] + ea[:, 1:2] * wea_ref[1:2, :])
    m = _silu(pre)
    o_ref[...] = _silu(_dot(m, w2_ref[...]) + b2_ref[...])


def _node_mid_body(x_ref, pa_ref, pb_ref, wx_ref, wa_ref, b1_ref, w2_ref,
                   b2_ref, ws_ref, wd_ref, o_ref, os_ref, od_ref):
    x = x_ref[...]
    agg = pa_ref[0] + pa_ref[1] + pb_ref[0] + pb_ref[1]
    t = _silu(_dot(x, wx_ref[...]) + _dot(agg, wa_ref[...]) + b1_ref[...])
    xn = x + _dot(t, w2_ref[...]) + b2_ref[...]
    o_ref[...] = xn
    os_ref[...] = _dot(xn, ws_ref[...])
    od_ref[...] = _dot(xn, wd_ref[...])


def _node_last_body(x_ref, pa_ref, pb_ref, wx_ref, wa_ref, b1_ref, w2_ref,
                    b2_ref, d1_ref, db1_ref, d2_ref, db2_ref, o_ref):
    x = x_ref[...]
    agg = pa_ref[0] + pa_ref[1] + pb_ref[0] + pb_ref[1]
    t = _silu(_dot(x, wx_ref[...]) + _dot(agg, wa_ref[...]) + b1_ref[...])
    xn = x + _dot(t, w2_ref[...]) + b2_ref[...]
    u = _silu(_dot(xn, d1_ref[...]) + db1_ref[...])
    o_ref[...] = _dot(u, d2_ref[...]) + db2_ref[...]


def _row_block(bn):
    return pl.BlockSpec((bn, H), lambda i: (i, 0))


def _full(shape):
    return pl.BlockSpec(shape, lambda i: tuple(0 for _ in shape))


def _embed(h, w, b, ws, wd):
    return pl.pallas_call(
        _embed_body,
        grid=(N // BN,),
        in_specs=[pl.BlockSpec((BN, D), lambda i: (i, 0)),
                  _full((D, H)), _full((1, H)), _full((H, H)), _full((H, H))],
        out_specs=[_row_block(BN), _row_block(BN), _row_block(BN)],
        out_shape=[jax.ShapeDtypeStruct((N, H), _f32)] * 3,
    )(h, w, b.reshape(1, H), ws, wd)


def _edge_mlp(g, ea, wea, b1, w2, b2):
    return pl.pallas_call(
        _edge_body,
        grid=(ES // BE,),
        in_specs=[_row_block(BE),
                  pl.BlockSpec((BE, DE), lambda i: (i, 0)),
                  _full((DE, H)), _full((1, H)), _full((H, H)), _full((1, H))],
        out_specs=_row_block(BE),
        out_shape=jax.ShapeDtypeStruct((ES, H), _f32),
    )(g, ea, wea, b1.reshape(1, H), w2, b2.reshape(1, H))


def _parts_specs():
    return [pl.BlockSpec((NC, BN, H), lambda i: (0, i, 0)),
            pl.BlockSpec((NC, BN, H), lambda i: (0, i, 0))]


def _node_mid(x, pa, pb, wx, wa, b1, w2, b2, ws, wd):
    return pl.pallas_call(
        _node_mid_body,
        grid=(N // BN,),
        in_specs=[_row_block(BN)] + _parts_specs() +
                 [_full((H, H)), _full((H, H)), _full((1, H)),
                  _full((H, H)), _full((1, H)), _full((H, H)), _full((H, H))],
        out_specs=[_row_block(BN), _row_block(BN), _row_block(BN)],
        out_shape=[jax.ShapeDtypeStruct((N, H), _f32)] * 3,
    )(x, pa, pb, wx, wa, b1.reshape(1, H), w2, b2.reshape(1, H), ws, wd)


def _node_last(x, pa, pb, wx, wa, b1, w2, b2, d1W, d1b, d2W, d2b):
    d2p = jnp.zeros((H, 128), _f32).at[:, :3].set(d2W)
    db2p = jnp.zeros((1, 128), _f32).at[0, :3].set(d2b)
    y = pl.pallas_call(
        _node_last_body,
        grid=(N // BN,),
        in_specs=[_row_block(BN)] + _parts_specs() +
                 [_full((H, H)), _full((H, H)), _full((1, H)),
                  _full((H, H)), _full((1, H)), _full((H, H)), _full((1, H)),
                  _full((H, 128)), _full((1, 128))],
        out_specs=pl.BlockSpec((BN, 128), lambda i: (i, 0)),
        out_shape=jax.ShapeDtypeStruct((N, 128), _f32),
    )(x, pa, pb, wx, wa, b1.reshape(1, H), w2, b2.reshape(1, H),
      d1W, d1b.reshape(1, H), d2p, db2p)
    return y[:, :3]


def _worker_range(wid):
    c0 = wid * MAXC
    cnt = jnp.minimum(MAXC, NCHUNK - c0)
    return c0, cnt


def _sc_gather_body(xs_hbm, xd_hbm, row_hbm, col_hbm, g_hbm,
                    ridx, cidx, ident, b1a, b1b, b2a, b2b, shared,
                    s1a, s1b, s2a, s2b, sw):
    cid = lax.axis_index("c")
    sid = lax.axis_index("s")
    wid = sid * NC + cid
    c0, cnt = _worker_range(wid)
    pltpu.sync_copy(row_hbm.at[pl.ds(c0, MAXC)], ridx)
    pltpu.sync_copy(col_hbm.at[pl.ds(c0, MAXC)], cidx)

    @pl.loop(0, CH // 16)
    def _(j):
        ia = lax.broadcasted_iota(_i32, (16,), 0) + j * 16
        ident[pl.ds(j * 16, 16)] = ia + sid * CH

    def fire(k, buf1, buf2, sem1, sem2):
        @pl.when(k < cnt)
        def _():
            pltpu.make_async_copy(xs_hbm.at[ridx.at[k]], buf1, sem1).start()
            pltpu.make_async_copy(xd_hbm.at[cidx.at[k]], buf2, sem2).start()

    def wb(k):
        base = (c0 + k) * CH
        return pltpu.make_async_copy(shared.at[pl.ds(sid * CH, CH)],
                                     g_hbm.at[pl.ds(base, CH)], sw)

    def drain(k, buf1, buf2, sem1, sem2):
        @pl.when(k < cnt)
        def _():
            r0 = sid * CH
            pltpu.make_async_copy(xs_hbm.at[ridx.at[k]], buf1, sem1).wait()
            pltpu.make_async_copy(xd_hbm.at[cidx.at[k]], buf2, sem2).wait()

            @pl.when(k > 0)
            def _():
                wb(k - 1).wait()

            pltpu.sync_copy(buf1, shared.at[pl.ds(r0, CH)])
            pltpu.sync_copy(buf2, shared.at[ident], add=True)
            wb(k).start()

    fire(0, b1a, b2a, s1a, s2a)

    @pl.loop(0, (MAXC + 1) // 2)
    def _(j):
        k = 2 * j
        fire(k + 1, b1b, b2b, s1b, s2b)
        drain(k, b1a, b2a, s1a, s2a)
        fire(k + 2, b1a, b2a, s1a, s2a)
        drain(k + 1, b1b, b2b, s1b, s2b)

    @pl.when(cnt > 0)
    def _():
        wb(cnt - 1).wait()


def _sc_gather(xs, xd, row2d, col2d):
    k = pl.kernel(
        _sc_gather_body,
        out_type=jax.ShapeDtypeStruct((ES, H), _f32),
        mesh=_mesh,
        scratch_types=[pltpu.VMEM((MAXC, CH), _i32),
                       pltpu.VMEM((MAXC, CH), _i32),
                       pltpu.VMEM((CH,), _i32),
                       pltpu.VMEM((CH, H), _f32),
                       pltpu.VMEM((CH, H), _f32),
                       pltpu.VMEM((CH, H), _f32),
                       pltpu.VMEM((CH, H), _f32),
                       pltpu.VMEM_SHARED((NS * CH, H), _f32),
                       pltpu.SemaphoreType.DMA,
                       pltpu.SemaphoreType.DMA,
                       pltpu.SemaphoreType.DMA,
                       pltpu.SemaphoreType.DMA,
                       pltpu.SemaphoreType.DMA],
    )
    return k(xs, xd, row2d, col2d)


def _sc_scatter_body(m2_hbm, row_hbm, out_hbm, ridx, bufa, bufb, zbuf, acc,
                     sema, semb, semz):
    cid = lax.axis_index("c")
    sid = lax.axis_index("s")
    wid = sid * NC + cid
    c0 = wid * MAXCS
    cnt = jnp.minimum(MAXCS, NCHUNKS - c0)

    pltpu.sync_copy(row_hbm.at[pl.ds(c0, MAXCS)], ridx)

    @pl.loop(0, ZR)
    def _(r):
        @pl.loop(0, H // 16)
        def _(j):
            zbuf[r, pl.ds(j * 16, 16)] = jnp.zeros((16,), _f32)

    def zcopy(j):
        z = sid + j * NS
        return z < NZC, pltpu.make_async_copy(zbuf, acc.at[pl.ds(z * ZR, ZR)],
                                              semz)

    @pl.loop(0, ZPW)
    def _(j):
        ok, cp = zcopy(j)

        @pl.when(ok)
        def _():
            cp.start()

    @pl.loop(0, ZPW)
    def _(j):
        ok, cp = zcopy(j)

        @pl.when(ok)
        def _():
            cp.wait()

    plsc.subcore_barrier()

    def fire(k, buf, sem):
        @pl.when(k < cnt)
        def _():
            base = (c0 + k) * CHS
            pltpu.make_async_copy(m2_hbm.at[pl.ds(base, CHS)], buf, sem).start()

    def drain(k, buf, sem):
        @pl.when(k < cnt)
        def _():
            base = (c0 + k) * CHS
            pltpu.make_async_copy(m2_hbm.at[pl.ds(base, CHS)], buf, sem).wait()
            pltpu.sync_copy(buf, acc.at[ridx.at[k]], add=True)

    fire(0, bufa, sema)

    @pl.loop(0, (MAXCS + 1) // 2)
    def _(j):
        k = 2 * j
        fire(k + 1, bufb, semb)
        drain(k, bufa, sema)
        fire(k + 2, bufa, sema)
        drain(k + 1, bufb, semb)

    plsc.subcore_barrier()

    def ocopy(j):
        z = sid + j * NS
        r0 = z * ZR
        return z < NZC, pltpu.make_async_copy(
            acc.at[pl.ds(r0, ZR)], out_hbm.at[cid, pl.ds(r0, ZR)], semz)

    @pl.loop(0, ZPW)
    def _(j):
        ok, cp = ocopy(j)

        @pl.when(ok)
        def _():
            cp.start()

    @pl.loop(0, ZPW)
    def _(j):
        ok, cp = ocopy(j)

        @pl.when(ok)
        def _():
            cp.wait()


def _sc_scatter(m2, row2d):
    k = pl.kernel(
        _sc_scatter_body,
        out_type=jax.ShapeDtypeStruct((NC, N, H), _f32),
        mesh=_mesh,
        scratch_types=[pltpu.VMEM((MAXCS, CHS), _i32),
                       pltpu.VMEM((CHS, H), _f32),
                       pltpu.VMEM((CHS, H), _f32),
                       pltpu.VMEM((ZR, H), _f32),
                       pltpu.VMEM_SHARED((N, H), _f32),
                       pltpu.SemaphoreType.DMA,
                       pltpu.SemaphoreType.DMA,
                       pltpu.SemaphoreType.DMA],
    )
    return k(m2, row2d)


def _slice_chunks(idx1d, ch, nchunk, ncpad):
    c = idx1d.reshape(NSL, nchunk, ch)
    pad = ((0, ncpad - nchunk), (0, 0))
    return [jnp.pad(c[s], pad) for s in range(NSL)]


def kernel(h, edges, edge_attr, emb_W, emb_b, e1W, e1b, e2W, e2b,
           n1W, n1b, n2W, n2b, d1W, d1b, d2W, d2b):
    rows = _slice_chunks(edges[0], CH, NCHUNK, NCPAD)
    cols = _slice_chunks(edges[1], CH, NCHUNK, NCPAD)
    rows_s = _slice_chunks(edges[0], CHS, NCHUNKS, NCPADS)
    eas = [edge_attr[s * ES:(s + 1) * ES] for s in range(NSL)]

    x, xs, xd = _embed(h, emb_W, emb_b, e1W[0, :H], e1W[0, H:2 * H])
    for i in range(NL):
        wea = e1W[i, 2 * H:]
        gs = [_sc_gather(xs, xd, rows[s], cols[s]) for s in range(NSL)]
        m2s = [_edge_mlp(gs[s], eas[s], wea, e1b[i], e2W[i], e2b[i])
               for s in range(NSL)]
        parts = [_sc_scatter(m2s[s], rows_s[s]) for s in range(NSL)]
        if i < NL - 1:
            x, xs, xd = _node_mid(x, parts[0], parts[1], n1W[i, :H],
                                  n1W[i, H:], n1b[i], n2W[i], n2b[i],
                                  e1W[i + 1, :H], e1W[i + 1, H:2 * H])
        else:
            return _node_last(x, parts[0], parts[1], n1W[i, :H], n1W[i, H:],
                              n1b[i], n2W[i], n2b[i], d1W, d1b, d2W, d2b)
